# baseline (speedup 1.0000x reference)
import functools
import jax
import jax.numpy as jnp
from jax import lax
from jax.experimental import pallas as pl
from jax.experimental.pallas import tpu as pltpu
from jax.experimental.pallas import tpu_sc as plsc

N = 4096
E = 16384
NG = 128
HTRUE = 300
HP = 384
DP = 768
F32 = jnp.float32


NS = 48
RS = 8


def _to_t3(h):
    return h.T.reshape(NS, RS, h.shape[0])


def _proj_kernel(a_ref, w_ref, b_ref, o_ref, t3_ref, *, full, t3):
    acc = jnp.dot(a_ref[...], w_ref[...], preferred_element_type=F32)
    h = jnp.maximum(acc + b_ref[...], 0.0)
    if full:
        o_ref[...] = h
    if t3:
        t3_ref[...] = _to_t3(h)


def _proj(a, wt, b, bm, *, full=True, t3=True):
    m, k = a.shape
    n = wt.shape[1]
    grid = m // bm
    out_specs, out_shape = [], []
    if full:
        out_specs.append(pl.BlockSpec((bm, n), lambda i: (i, 0)))
        out_shape.append(jax.ShapeDtypeStruct((m, n), F32))
    if t3:
        out_specs.append(pl.BlockSpec((NS, RS, bm), lambda i: (0, 0, i)))
        out_shape.append(jax.ShapeDtypeStruct((NS, RS, m), F32))

    def kern(a_ref, w_ref, b_ref, *outs):
        o = outs[0] if full else None
        t = outs[-1] if t3 else None
        _proj_kernel(a_ref, w_ref, b_ref, o, t, full=full, t3=t3)

    res = pl.pallas_call(
        kern,
        grid=(grid,),
        in_specs=[
            pl.BlockSpec((bm, k), lambda i: (i, 0)),
            pl.BlockSpec((k, n), lambda i: (0, 0)),
            pl.BlockSpec((1, n), lambda i: (0, 0)),
        ],
        out_specs=out_specs,
        out_shape=out_shape,
    )(a, wt, b)
    return res if len(res) > 1 else res[0]


def _mlp_kernel(h_ref, a3_ref, w1_ref, b1_ref, w2_ref, b2_ref,
                *outs, l2norm, emit_t, t3):
    bm = h_ref.shape[0]
    agg = a3_ref[...].reshape(HP, bm).T
    x = h_ref[...] + agg
    z = jnp.maximum(jnp.dot(x, w1_ref[...], preferred_element_type=F32)
                    + b1_ref[...], 0.0)
    h = jnp.maximum(jnp.dot(z, w2_ref[...], preferred_element_type=F32)
                    + b2_ref[...], 0.0)
    if l2norm:
        nrm = jnp.sqrt(jnp.sum(h * h, axis=1, keepdims=True))
        h = h / jnp.maximum(nrm, 1e-12)
    outs[0][...] = h
    if t3:
        outs[1][...] = _to_t3(h)
    if emit_t:
        outs[-1][...] = h.T


def _mlp(h, a3, w1t, b1, w2t, b2, *, l2norm, emit_t, t3, bm=512):
    m, n = h.shape
    grid = m // bm
    out_shape = [jax.ShapeDtypeStruct((m, n), F32)]
    out_specs = [pl.BlockSpec((bm, n), lambda i: (i, 0))]
    if t3:
        out_shape.append(jax.ShapeDtypeStruct((NS, RS, m), F32))
        out_specs.append(pl.BlockSpec((NS, RS, bm), lambda i: (0, 0, i)))
    if emit_t:
        out_shape.append(jax.ShapeDtypeStruct((n, m), F32))
        out_specs.append(pl.BlockSpec((n, bm), lambda i: (0, i)))
    res = pl.pallas_call(
        functools.partial(_mlp_kernel, l2norm=l2norm, emit_t=emit_t, t3=t3),
        grid=(grid,),
        in_specs=[
            pl.BlockSpec((bm, n), lambda i: (i, 0)),
            pl.BlockSpec((NS, RS, bm), lambda i: (0, 0, i)),
            pl.BlockSpec((n, n), lambda i: (0, 0)),
            pl.BlockSpec((1, n), lambda i: (0, 0)),
            pl.BlockSpec((n, n), lambda i: (0, 0)),
            pl.BlockSpec((1, n), lambda i: (0, 0)),
        ],
        out_specs=out_specs,
        out_shape=out_shape,
    )(h, a3, w1t, b1, w2t, b2)
    return res if len(res) > 1 else res[0]


_SCN = 512
_NSC = E // _SCN
_ATILES = NS // 2
_LIVE = 38


def _sc_msg_body(h3, e3, src_hbm, dst_hbm, out_hbm,
                 src_st, dst_st, e_buf, h_slab, acc, sem):
    wid = lax.axis_index("s") * 2 + lax.axis_index("c")

    @pl.when(wid < _ATILES)
    def _():
        for si in range(2):
            slab = wid * 2 + si

            def _zero(i, _):
                for r in range(RS):
                    acc[r, pl.ds(i * 16, 16)] = jnp.zeros((16,), F32)
                return _
            lax.fori_loop(0, N // 16, _zero, None)

            @pl.when(slab < _LIVE)
            def _():
                pltpu.sync_copy(h3.at[slab], h_slab)

                def _start(cb, b):
                    base = cb * _SCN
                    pltpu.async_copy(src_hbm.at[pl.ds(base, _SCN)],
                                     src_st.at[b], sem.at[b])
                    pltpu.async_copy(dst_hbm.at[pl.ds(base, _SCN)],
                                     dst_st.at[b], sem.at[b])
                    pltpu.async_copy(e3.at[slab, :, pl.ds(base, _SCN)],
                                     e_buf.at[b], sem.at[b])

                def _wait(cb, b):
                    base = cb * _SCN
                    pltpu.make_async_copy(src_hbm.at[pl.ds(base, _SCN)],
                                          src_st.at[b], sem.at[b]).wait()
                    pltpu.make_async_copy(dst_hbm.at[pl.ds(base, _SCN)],
                                          dst_st.at[b], sem.at[b]).wait()
                    pltpu.make_async_copy(e3.at[slab, :, pl.ds(base, _SCN)],
                                          e_buf.at[b], sem.at[b]).wait()

                _start(0, 0)

                def _chunk(cb, _):
                    b = lax.rem(cb, 2)

                    @pl.when(cb + 1 < _NSC)
                    def _():
                        _start(cb + 1, 1 - b)
                    _wait(cb, b)

                    def _grp(g, _):
                        for jj in range(2):
                            j = g * 2 + jj
                            s16 = src_st[b, pl.ds(j * 16, 16)]
                            d16 = dst_st[b, pl.ds(j * 16, 16)]
                            for r in range(RS):
                                rv = jnp.full((16,), r, jnp.int32)
                                hv = plsc.load_gather(h_slab, [rv, s16])
                                ev = e_buf[b, r, pl.ds(j * 16, 16)]
                                msg = jnp.maximum(hv + ev, 0.0)
                                plsc.addupdate_scatter(acc, [rv, d16], msg)
                        return _
                    lax.fori_loop(0, _SCN // 32, _grp, None)
                    return _
                lax.fori_loop(0, _NSC, _chunk, None)

            pltpu.sync_copy(acc, out_hbm.at[slab])


def _sc_msg(h3, e3, src, dst):
    mesh = plsc.VectorSubcoreMesh(core_axis_name="c", subcore_axis_name="s")
    return pl.kernel(
        _sc_msg_body,
        out_type=jax.ShapeDtypeStruct((NS, RS, N), F32),
        mesh=mesh,
        compiler_params=pltpu.CompilerParams(needs_layout_passes=False),
        scratch_types=[
            pltpu.VMEM((2, _SCN), jnp.int32),
            pltpu.VMEM((2, _SCN), jnp.int32),
            pltpu.VMEM((2, RS, _SCN), F32),
            pltpu.VMEM((RS, N), F32),
            pltpu.VMEM((RS, N), F32),
            pltpu.SemaphoreType.DMA((2,)),
        ],
    )(h3, e3, src, dst)


def _inter_kernel(u_ref, vt_ref, v_ref, bu_ref, bv_ref,
                  imap_ref, up_ref, vp_ref):
    i = pl.program_id(0)

    @pl.when(i == 0)
    def _():
        vp_ref[...] = jnp.zeros_like(vp_ref)

    imap_ref[...] = jnp.zeros_like(imap_ref)

    bu = bu_ref[...]
    bv = bv_ref[...]
    blo = jnp.min(bu)
    bhi = jnp.max(bu)
    start = jnp.sum((bv < blo).astype(F32)).astype(jnp.int32)
    end = jnp.sum((bv <= bhi).astype(F32)).astype(jnp.int32)
    bm = 128
    jlo = start // bm
    jhi = (end + bm - 1) // bm

    u_blk = u_ref[...]

    def _col(j, acc):
        c0 = j * bm
        s = jnp.dot(u_blk, vt_ref[:, pl.ds(c0, bm)],
                    preferred_element_type=F32)
        mask = bu == bv_ref[0:1, pl.ds(c0, bm)]
        s = jnp.where(mask, s, 0.0)
        imap_ref[:, pl.ds(c0, bm)] = s
        acc = acc + jnp.dot(s, v_ref[pl.ds(c0, bm), :],
                            preferred_element_type=F32)
        vp_ref[pl.ds(c0, bm), :] += lax.dot_general(
            s, u_blk, (((0,), (0,)), ((), ())),
            preferred_element_type=F32)
        return acc
    up_ref[...] = lax.fori_loop(jlo, jhi, _col, jnp.zeros((bm, HP), F32))


def _interact(u, v, vt, bu, bv):
    bm = 128
    grid = N // bm
    return pl.pallas_call(
        _inter_kernel,
        grid=(grid,),
        in_specs=[
            pl.BlockSpec((bm, HP), lambda i: (i, 0)),
            pl.BlockSpec((HP, N), lambda i: (0, 0)),
            pl.BlockSpec((N, HP), lambda i: (0, 0)),
            pl.BlockSpec((bm, 1), lambda i: (i, 0)),
            pl.BlockSpec((1, N), lambda i: (0, 0)),
        ],
        out_specs=[
            pl.BlockSpec((bm, N), lambda i: (i, 0)),
            pl.BlockSpec((bm, HP), lambda i: (i, 0)),
            pl.BlockSpec((N, HP), lambda i: (0, 0)),
        ],
        out_shape=[
            jax.ShapeDtypeStruct((N, N), F32),
            jax.ShapeDtypeStruct((N, HP), F32),
            jax.ShapeDtypeStruct((N, HP), F32),
        ],
    )(u, vt, v, bu, bv)


def _s2s_iter_kernel(x_ref, mf_ref, q_ref, h_ref, c_ref,
                     wih_ref, whh_ref, bih_ref, bhh_ref,
                     qs_ref, ho_ref, co_ref):
    gates = (jnp.dot(q_ref[...], wih_ref[...], preferred_element_type=F32)
             + bih_ref[...]
             + jnp.dot(h_ref[...], whh_ref[...], preferred_element_type=F32)
             + bhh_ref[...])
    ig = jax.nn.sigmoid(gates[:, 0 * DP:1 * DP])
    fg = jax.nn.sigmoid(gates[:, 1 * DP:2 * DP])
    gg = jnp.tanh(gates[:, 2 * DP:3 * DP])
    og = jax.nn.sigmoid(gates[:, 3 * DP:4 * DP])
    cc = fg * c_ref[...] + ig * gg
    q = og * jnp.tanh(cc)
    ho_ref[...] = q
    co_ref[...] = cc
    x = x_ref[...]
    mf = mf_ref[...]
    escore = lax.dot_general(x, q, (((1,), (1,)), ((), ())),
                             preferred_element_type=F32)
    e_node = jnp.sum(escore * mf, axis=1, keepdims=True)
    em = e_node * mf + (mf - 1.0) * 1e30
    emax = jnp.max(em, axis=0, keepdims=True)
    e_max_node = jnp.sum(mf * emax, axis=1, keepdims=True)
    a = jnp.exp(e_node - e_max_node)
    asum = jnp.sum(a * mf, axis=0, keepdims=True)
    asum_node = jnp.sum(mf * asum, axis=1, keepdims=True)
    a = a / (asum_node + 1e-16)
    r = lax.dot_general(mf, a * x, (((0,), (0,)), ((), ())),
                        preferred_element_type=F32)
    qs_ref[...] = jnp.concatenate([q, r], axis=1)


def _set2set(x, mf, wih, whh, bih, bhh, steps):
    q_star = jnp.zeros((NG, 2 * DP), F32)
    hh = jnp.zeros((NG, DP), F32)
    cc = jnp.zeros((NG, DP), F32)
    for _ in range(steps):
        q_star, hh, cc = pl.pallas_call(
            _s2s_iter_kernel,
            out_shape=[jax.ShapeDtypeStruct((NG, 2 * DP), F32),
                       jax.ShapeDtypeStruct((NG, DP), F32),
                       jax.ShapeDtypeStruct((NG, DP), F32)],
        )(x, mf, q_star, hh, cc, wih, whh, bih, bhh)
    return q_star


def _pred_kernel(f_ref, w_ref, b_ref, o_ref):
    o_ref[...] = (jnp.dot(f_ref[...], w_ref[...], preferred_element_type=F32)
                  + b_ref[...])


def _pred(f, wt, b):
    return pl.pallas_call(
        _pred_kernel,
        out_shape=jax.ShapeDtypeStruct((NG, 128), F32),
    )(f, wt, b)


def _padt(w, rows, cols):
    wt = w.T
    return jnp.zeros((rows, cols), F32).at[:wt.shape[0], :wt.shape[1]].set(wt)


def _pad_vec(b, n):
    return jnp.zeros((1, n), F32).at[0, :b.shape[0]].set(b)


def _prep(params):
    p = {}
    p['w0t'] = _padt(params['lin0_w'], 133, HP)
    p['b0'] = _pad_vec(params['lin0_b'], HP)
    p['wet'] = _padt(params['edge_w'], 14, HP)
    p['be'] = _pad_vec(params['edge_b'], HP)
    for i in range(3):
        p['w1t%d' % i] = _padt(params['mlp1_w_%d' % i], HP, HP)
        p['b1%d' % i] = _pad_vec(params['mlp1_b_%d' % i], HP)
        p['w2t%d' % i] = _padt(params['mlp2_w_%d' % i], HP, HP)
        p['b2%d' % i] = _pad_vec(params['mlp2_b_%d' % i], HP)
    half = ((0, 0), (300, 384))
    qs_half = ((0, 0), (300, 384), (600, DP), (900, DP + 384))
    wih = params['lstm_wih']
    whh = params['lstm_whh']
    wih_p = jnp.zeros((2 * DP, 4 * DP), F32)
    whh_p = jnp.zeros((DP, 4 * DP), F32)
    bih_p = jnp.zeros((1, 4 * DP), F32)
    bhh_p = jnp.zeros((1, 4 * DP), F32)
    for g in range(4):
        for (to, po) in half:
            ro = g * 600 + to
            co = g * DP + po
            for (ti, pi) in qs_half:
                wih_p = wih_p.at[pi:pi + 300, co:co + 300].set(
                    wih[ro:ro + 300, ti:ti + 300].T)
            for (ti, pi) in half:
                whh_p = whh_p.at[pi:pi + 300, co:co + 300].set(
                    whh[ro:ro + 300, ti:ti + 300].T)
            bih_p = bih_p.at[0, co:co + 300].set(
                params['lstm_bih'][ro:ro + 300])
            bhh_p = bhh_p.at[0, co:co + 300].set(
                params['lstm_bhh'][ro:ro + 300])
    p['wih'] = wih_p
    p['whh'] = whh_p
    p['bih'] = bih_p
    p['bhh'] = bhh_p
    pw = params['pred_w'][0]
    pwt = jnp.zeros((4 * DP, 128), F32)
    for g in range(4):
        for (to, po) in half:
            pwt = pwt.at[g * DP + po:g * DP + po + 300, 0].set(
                pw[g * 600 + to:g * 600 + to + 300])
    p['predwt'] = pwt
    p['predb'] = jnp.zeros((1, 128), F32).at[0, 0].set(params['pred_b'][0])
    return p


def _gine_side(x, eattr, src, dst, pp, *, emit_t):
    h, h3 = _proj(x, pp['w0t'], pp['b0'], 512)
    e3 = _proj(eattr, pp['wet'], pp['be'], 1024, full=False)
    for i in range(3):
        a3 = _sc_msg(h3, e3, src, dst)
        last = (i == 2)
        res = _mlp(h, a3,
                   pp['w1t%d' % i], pp['b1%d' % i],
                   pp['w2t%d' % i], pp['b2%d' % i],
                   l2norm=last, emit_t=(last and emit_t), t3=not last)
        if last:
            return res
        h, h3 = res


def kernel(solute_x, solute_edge_index, solute_edge_attr, solute_batch,
           solute_len, solvent_x, solvent_edge_index, solvent_edge_attr,
           solvent_batch, solvent_len, params):
    pp = _prep(params)
    src_u = solute_edge_index[0].astype(jnp.int32)
    dst_u = solute_edge_index[1].astype(jnp.int32)
    src_v = solvent_edge_index[0].astype(jnp.int32)
    dst_v = solvent_edge_index[1].astype(jnp.int32)

    u = _gine_side(solute_x, solute_edge_attr, src_u, dst_u, pp, emit_t=False)
    v, vt = _gine_side(solvent_x, solvent_edge_attr, src_v, dst_v, pp,
                       emit_t=True)

    bu = solute_batch.astype(F32).reshape(N, 1)
    bv = solvent_batch.astype(F32).reshape(1, N)
    imap, u_p, v_p = _interact(u, v, vt, bu, bv)

    x_u = jnp.concatenate([u, u_p], axis=1)
    x_v = jnp.concatenate([v, v_p], axis=1)
    mf_u = solute_len.T
    mf_v = solvent_len.T
    qs_u = _set2set(x_u, mf_u, pp['wih'], pp['whh'], pp['bih'], pp['bhh'], 2)
    qs_v = _set2set(x_v, mf_v, pp['wih'], pp['whh'], pp['bih'], pp['bhh'], 2)

    final = jnp.concatenate([qs_u, qs_v], axis=1)
    pred = _pred(final, pp['predwt'], pp['predb'])[:, :1]
    return pred, imap


# speedup vs baseline: 1.1254x; 1.1254x over previous
import functools
import jax
import jax.numpy as jnp
from jax import lax
from jax.experimental import pallas as pl
from jax.experimental.pallas import tpu as pltpu
from jax.experimental.pallas import tpu_sc as plsc

N = 4096
E = 16384
NG = 128
HTRUE = 300
HP = 384
DP = 768
F32 = jnp.float32


NS = 48
RS = 8


def _to_t3(h):
    return h.T.reshape(NS, RS, h.shape[0])


def _proj_kernel(a_ref, w_ref, b_ref, o_ref, t3_ref, *, full, t3):
    acc = jnp.dot(a_ref[...], w_ref[...], preferred_element_type=F32)
    h = jnp.maximum(acc + b_ref[...], 0.0)
    if full:
        o_ref[...] = h
    if t3:
        t3_ref[...] = _to_t3(h)


def _proj(a, wt, b, bm, *, full=True, t3=True):
    m, k = a.shape
    n = wt.shape[1]
    grid = m // bm
    out_specs, out_shape = [], []
    if full:
        out_specs.append(pl.BlockSpec((bm, n), lambda i: (i, 0)))
        out_shape.append(jax.ShapeDtypeStruct((m, n), F32))
    if t3:
        out_specs.append(pl.BlockSpec((NS, RS, bm), lambda i: (0, 0, i)))
        out_shape.append(jax.ShapeDtypeStruct((NS, RS, m), F32))

    def kern(a_ref, w_ref, b_ref, *outs):
        o = outs[0] if full else None
        t = outs[-1] if t3 else None
        _proj_kernel(a_ref, w_ref, b_ref, o, t, full=full, t3=t3)

    res = pl.pallas_call(
        kern,
        grid=(grid,),
        in_specs=[
            pl.BlockSpec((bm, k), lambda i: (i, 0)),
            pl.BlockSpec((k, n), lambda i: (0, 0)),
            pl.BlockSpec((1, n), lambda i: (0, 0)),
        ],
        out_specs=out_specs,
        out_shape=out_shape,
    )(a, wt, b)
    return res if len(res) > 1 else res[0]


def _mlp_kernel(h_ref, a3_ref, p2_ref, w1_ref, b1_ref, w2_ref, b2_ref,
                *outs, l2norm, emit_t, t3):
    bm = h_ref.shape[0]
    a = a3_ref[...]
    p = p2_ref[...]
    psum = p[0:6] + p[6:12] + p[12:18] + p[18:24]
    a = jnp.concatenate([a[:32], a[32:38] + psum, a[38:]], axis=0)
    agg = a.reshape(HP, bm).T
    x = h_ref[...] + agg
    z = jnp.maximum(jnp.dot(x, w1_ref[...], preferred_element_type=F32)
                    + b1_ref[...], 0.0)
    h = jnp.maximum(jnp.dot(z, w2_ref[...], preferred_element_type=F32)
                    + b2_ref[...], 0.0)
    if l2norm:
        nrm = jnp.sqrt(jnp.sum(h * h, axis=1, keepdims=True))
        h = h / jnp.maximum(nrm, 1e-12)
    outs[0][...] = h
    if t3:
        outs[1][...] = _to_t3(h)
    if emit_t:
        outs[-1][...] = h.T


def _mlp(h, a3, p2, w1t, b1, w2t, b2, *, l2norm, emit_t, t3, bm=512):
    m, n = h.shape
    grid = m // bm
    out_shape = [jax.ShapeDtypeStruct((m, n), F32)]
    out_specs = [pl.BlockSpec((bm, n), lambda i: (i, 0))]
    if t3:
        out_shape.append(jax.ShapeDtypeStruct((NS, RS, m), F32))
        out_specs.append(pl.BlockSpec((NS, RS, bm), lambda i: (0, 0, i)))
    if emit_t:
        out_shape.append(jax.ShapeDtypeStruct((n, m), F32))
        out_specs.append(pl.BlockSpec((n, bm), lambda i: (0, i)))
    res = pl.pallas_call(
        functools.partial(_mlp_kernel, l2norm=l2norm, emit_t=emit_t, t3=t3),
        grid=(grid,),
        in_specs=[
            pl.BlockSpec((bm, n), lambda i: (i, 0)),
            pl.BlockSpec((NS, RS, bm), lambda i: (0, 0, i)),
            pl.BlockSpec((24, RS, bm), lambda i: (0, 0, i)),
            pl.BlockSpec((n, n), lambda i: (0, 0)),
            pl.BlockSpec((1, n), lambda i: (0, 0)),
            pl.BlockSpec((n, n), lambda i: (0, 0)),
            pl.BlockSpec((1, n), lambda i: (0, 0)),
        ],
        out_specs=out_specs,
        out_shape=out_shape,
    )(h, a3, p2, w1t, b1, w2t, b2)
    return res if len(res) > 1 else res[0]


_SCN = 512
_NSC = E // _SCN
_ATILES = NS // 2
_LIVE = 38


def _sc_msg_body(h3, e3, src_hbm, dst_hbm, out_hbm, out2_hbm,
                 src_st, dst_st, e_buf, h_slab, acc, sem):
    wid = lax.axis_index("s") * 2 + lax.axis_index("c")

    def _zero_acc():
        def _zero(i, _):
            for r in range(RS):
                acc[r, pl.ds(i * 16, 16)] = jnp.zeros((16,), F32)
            return _
        lax.fori_loop(0, N // 16, _zero, None)

    def _edges(slab, lo, hi):
        pltpu.sync_copy(h3.at[slab], h_slab)

        def _start(cb, b):
            base = cb * _SCN
            pltpu.async_copy(src_hbm.at[pl.ds(base, _SCN)],
                             src_st.at[b], sem.at[b])
            pltpu.async_copy(dst_hbm.at[pl.ds(base, _SCN)],
                             dst_st.at[b], sem.at[b])
            pltpu.async_copy(e3.at[slab, :, pl.ds(base, _SCN)],
                             e_buf.at[b], sem.at[b])

        def _wait(cb, b):
            base = cb * _SCN
            pltpu.make_async_copy(src_hbm.at[pl.ds(base, _SCN)],
                                  src_st.at[b], sem.at[b]).wait()
            pltpu.make_async_copy(dst_hbm.at[pl.ds(base, _SCN)],
                                  dst_st.at[b], sem.at[b]).wait()
            pltpu.make_async_copy(e3.at[slab, :, pl.ds(base, _SCN)],
                                  e_buf.at[b], sem.at[b]).wait()

        _start(lo, lax.rem(lo, 2))

        def _chunk(cb, _):
            b = lax.rem(cb, 2)

            @pl.when(cb + 1 < hi)
            def _():
                _start(cb + 1, 1 - b)
            _wait(cb, b)

            def _grp(g, _):
                for jj in range(2):
                    j = g * 2 + jj
                    s16 = src_st[b, pl.ds(j * 16, 16)]
                    d16 = dst_st[b, pl.ds(j * 16, 16)]
                    for r in range(RS):
                        rv = jnp.full((16,), r, jnp.int32)
                        hv = plsc.load_gather(h_slab, [rv, s16])
                        ev = e_buf[b, r, pl.ds(j * 16, 16)]
                        msg = jnp.maximum(hv + ev, 0.0)
                        plsc.addupdate_scatter(acc, [rv, d16], msg)
                return _
            lax.fori_loop(0, _SCN // 32, _grp, None)
            return _
        lax.fori_loop(lo, hi, _chunk, None)

    _zero_acc()

    @pl.when(wid < NS - _LIVE)
    def _():
        pltpu.sync_copy(acc, out_hbm.at[_LIVE + wid])

    _edges(wid, 0, _NSC)
    pltpu.sync_copy(acc, out_hbm.at[wid])

    @pl.when((wid >= 6) & (wid < 30))
    def _():
        k = wid - 6
        slab2 = 32 + lax.rem(k, 6)
        q = k // 6
        _zero_acc()

        @pl.when(q == 0)
        def _():
            pltpu.sync_copy(acc, out_hbm.at[slab2])
        nq = _NSC // 4
        _edges(slab2, q * nq, (q + 1) * nq)
        pltpu.sync_copy(acc, out2_hbm.at[k])


def _sc_msg(h3, e3, src, dst):
    mesh = plsc.VectorSubcoreMesh(core_axis_name="c", subcore_axis_name="s")
    return pl.kernel(
        _sc_msg_body,
        out_type=[jax.ShapeDtypeStruct((NS, RS, N), F32),
                  jax.ShapeDtypeStruct((24, RS, N), F32)],
        mesh=mesh,
        compiler_params=pltpu.CompilerParams(needs_layout_passes=False),
        scratch_types=[
            pltpu.VMEM((2, _SCN), jnp.int32),
            pltpu.VMEM((2, _SCN), jnp.int32),
            pltpu.VMEM((2, RS, _SCN), F32),
            pltpu.VMEM((RS, N), F32),
            pltpu.VMEM((RS, N), F32),
            pltpu.SemaphoreType.DMA((2,)),
        ],
    )(h3, e3, src, dst)


def _inter_kernel(u_ref, vt_ref, v_ref, bu_ref, bv_ref,
                  imap_ref, up_ref, vp_ref):
    i = pl.program_id(0)

    @pl.when(i == 0)
    def _():
        vp_ref[...] = jnp.zeros_like(vp_ref)

    imap_ref[...] = jnp.zeros_like(imap_ref)

    bu = bu_ref[...]
    bv = bv_ref[...]
    blo = jnp.min(bu)
    bhi = jnp.max(bu)
    start = jnp.sum((bv < blo).astype(F32)).astype(jnp.int32)
    end = jnp.sum((bv <= bhi).astype(F32)).astype(jnp.int32)
    bm = 128
    jlo = start // bm
    jhi = (end + bm - 1) // bm

    u_blk = u_ref[...]

    def _col(j, acc):
        c0 = j * bm
        s = jnp.dot(u_blk, vt_ref[:, pl.ds(c0, bm)],
                    preferred_element_type=F32)
        mask = bu == bv_ref[0:1, pl.ds(c0, bm)]
        s = jnp.where(mask, s, 0.0)
        imap_ref[:, pl.ds(c0, bm)] = s
        acc = acc + jnp.dot(s, v_ref[pl.ds(c0, bm), :],
                            preferred_element_type=F32)
        vp_ref[pl.ds(c0, bm), :] += lax.dot_general(
            s, u_blk, (((0,), (0,)), ((), ())),
            preferred_element_type=F32)
        return acc
    up_ref[...] = lax.fori_loop(jlo, jhi, _col, jnp.zeros((bm, HP), F32))


def _interact(u, v, vt, bu, bv):
    bm = 128
    grid = N // bm
    return pl.pallas_call(
        _inter_kernel,
        grid=(grid,),
        in_specs=[
            pl.BlockSpec((bm, HP), lambda i: (i, 0)),
            pl.BlockSpec((HP, N), lambda i: (0, 0)),
            pl.BlockSpec((N, HP), lambda i: (0, 0)),
            pl.BlockSpec((bm, 1), lambda i: (i, 0)),
            pl.BlockSpec((1, N), lambda i: (0, 0)),
        ],
        out_specs=[
            pl.BlockSpec((bm, N), lambda i: (i, 0)),
            pl.BlockSpec((bm, HP), lambda i: (i, 0)),
            pl.BlockSpec((N, HP), lambda i: (0, 0)),
        ],
        out_shape=[
            jax.ShapeDtypeStruct((N, N), F32),
            jax.ShapeDtypeStruct((N, HP), F32),
            jax.ShapeDtypeStruct((N, HP), F32),
        ],
    )(u, vt, v, bu, bv)


def _lstm_kernel(q_ref, h_ref, c_ref, wih_ref, whh_ref, bih_ref, bhh_ref,
                 ho_ref, co_ref):
    gates = (jnp.dot(q_ref[...], wih_ref[...], preferred_element_type=F32)
             + bih_ref[...]
             + jnp.dot(h_ref[...], whh_ref[...], preferred_element_type=F32)
             + bhh_ref[...])
    ig = jax.nn.sigmoid(gates[:, 0 * DP:1 * DP])
    fg = jax.nn.sigmoid(gates[:, 1 * DP:2 * DP])
    gg = jnp.tanh(gates[:, 2 * DP:3 * DP])
    og = jax.nn.sigmoid(gates[:, 3 * DP:4 * DP])
    cc = fg * c_ref[...] + ig * gg
    ho_ref[...] = og * jnp.tanh(cc)
    co_ref[...] = cc


def _s2s_node_kernel(x_ref, mf_ref, q_ref, qs_ref):
    x = x_ref[...]
    mf = mf_ref[...]
    q = q_ref[...]
    escore = lax.dot_general(x, q, (((1,), (1,)), ((), ())),
                             preferred_element_type=F32,
                             precision=jax.lax.Precision.HIGHEST)
    e_node = jnp.sum(escore * mf, axis=1, keepdims=True)
    em = e_node * mf + (mf - 1.0) * 1e30
    emax = jnp.max(em, axis=0, keepdims=True)
    e_max_node = jnp.sum(mf * emax, axis=1, keepdims=True)
    a = jnp.exp(e_node - e_max_node)
    asum = jnp.sum(a * mf, axis=0, keepdims=True)
    asum_node = jnp.sum(mf * asum, axis=1, keepdims=True)
    a = a / (asum_node + 1e-16)
    r = lax.dot_general(mf, a * x, (((0,), (0,)), ((), ())),
                        preferred_element_type=F32)
    qs_ref[...] = jnp.concatenate([q, r], axis=1)


def _set2set(x, mf, wih, whh, bih, bhh, steps):
    q_star = jnp.zeros((NG, 2 * DP), F32)
    hh = jnp.zeros((NG, DP), F32)
    cc = jnp.zeros((NG, DP), F32)
    for _ in range(steps):
        hh, cc = pl.pallas_call(
            _lstm_kernel,
            out_shape=[jax.ShapeDtypeStruct((NG, DP), F32),
                       jax.ShapeDtypeStruct((NG, DP), F32)],
        )(q_star, hh, cc, wih, whh, bih, bhh)
        q_star = pl.pallas_call(
            _s2s_node_kernel,
            out_shape=jax.ShapeDtypeStruct((NG, 2 * DP), F32),
        )(x, mf, hh)
    return q_star


def _pred_kernel(f_ref, w_ref, b_ref, o_ref):
    o_ref[...] = (jnp.dot(f_ref[...], w_ref[...], preferred_element_type=F32)
                  + b_ref[...])


def _pred(f, wt, b):
    return pl.pallas_call(
        _pred_kernel,
        out_shape=jax.ShapeDtypeStruct((NG, 128), F32),
    )(f, wt, b)


def _padt(w, rows, cols):
    wt = w.T
    return jnp.zeros((rows, cols), F32).at[:wt.shape[0], :wt.shape[1]].set(wt)


def _pad_vec(b, n):
    return jnp.zeros((1, n), F32).at[0, :b.shape[0]].set(b)


def _prep(params):
    p = {}
    p['w0t'] = _padt(params['lin0_w'], 133, HP)
    p['b0'] = _pad_vec(params['lin0_b'], HP)
    p['wet'] = _padt(params['edge_w'], 14, HP)
    p['be'] = _pad_vec(params['edge_b'], HP)
    for i in range(3):
        p['w1t%d' % i] = _padt(params['mlp1_w_%d' % i], HP, HP)
        p['b1%d' % i] = _pad_vec(params['mlp1_b_%d' % i], HP)
        p['w2t%d' % i] = _padt(params['mlp2_w_%d' % i], HP, HP)
        p['b2%d' % i] = _pad_vec(params['mlp2_b_%d' % i], HP)
    half = ((0, 0), (300, 384))
    qs_half = ((0, 0), (300, 384), (600, DP), (900, DP + 384))
    wih = params['lstm_wih']
    whh = params['lstm_whh']
    wih_p = jnp.zeros((2 * DP, 4 * DP), F32)
    whh_p = jnp.zeros((DP, 4 * DP), F32)
    bih_p = jnp.zeros((1, 4 * DP), F32)
    bhh_p = jnp.zeros((1, 4 * DP), F32)
    for g in range(4):
        for (to, po) in half:
            ro = g * 600 + to
            co = g * DP + po
            for (ti, pi) in qs_half:
                wih_p = wih_p.at[pi:pi + 300, co:co + 300].set(
                    wih[ro:ro + 300, ti:ti + 300].T)
            for (ti, pi) in half:
                whh_p = whh_p.at[pi:pi + 300, co:co + 300].set(
                    whh[ro:ro + 300, ti:ti + 300].T)
            bih_p = bih_p.at[0, co:co + 300].set(
                params['lstm_bih'][ro:ro + 300])
            bhh_p = bhh_p.at[0, co:co + 300].set(
                params['lstm_bhh'][ro:ro + 300])
    p['wih'] = wih_p
    p['whh'] = whh_p
    p['bih'] = bih_p
    p['bhh'] = bhh_p
    pw = params['pred_w'][0]
    pwt = jnp.zeros((4 * DP, 128), F32)
    for g in range(4):
        for (to, po) in half:
            pwt = pwt.at[g * DP + po:g * DP + po + 300, 0].set(
                pw[g * 600 + to:g * 600 + to + 300])
    p['predwt'] = pwt
    p['predb'] = jnp.zeros((1, 128), F32).at[0, 0].set(params['pred_b'][0])
    return p


def _gine_side(x, eattr, src, dst, pp, *, emit_t):
    h, h3 = _proj(x, pp['w0t'], pp['b0'], 512)
    e3 = _proj(eattr, pp['wet'], pp['be'], 1024, full=False)
    for i in range(3):
        a3, p2 = _sc_msg(h3, e3, src, dst)
        last = (i == 2)
        res = _mlp(h, a3, p2,
                   pp['w1t%d' % i], pp['b1%d' % i],
                   pp['w2t%d' % i], pp['b2%d' % i],
                   l2norm=last, emit_t=(last and emit_t), t3=not last)
        if last:
            return res
        h, h3 = res


def kernel(solute_x, solute_edge_index, solute_edge_attr, solute_batch,
           solute_len, solvent_x, solvent_edge_index, solvent_edge_attr,
           solvent_batch, solvent_len, params):
    pp = _prep(params)
    src_u = solute_edge_index[0].astype(jnp.int32)
    dst_u = solute_edge_index[1].astype(jnp.int32)
    src_v = solvent_edge_index[0].astype(jnp.int32)
    dst_v = solvent_edge_index[1].astype(jnp.int32)

    u = _gine_side(solute_x, solute_edge_attr, src_u, dst_u, pp, emit_t=False)
    v, vt = _gine_side(solvent_x, solvent_edge_attr, src_v, dst_v, pp,
                       emit_t=True)

    bu = solute_batch.astype(F32).reshape(N, 1)
    bv = solvent_batch.astype(F32).reshape(1, N)
    imap, u_p, v_p = _interact(u, v, vt, bu, bv)

    x_u = jnp.concatenate([u, u_p], axis=1)
    x_v = jnp.concatenate([v, v_p], axis=1)
    mf_u = solute_len.T
    mf_v = solvent_len.T
    qs_u = _set2set(x_u, mf_u, pp['wih'], pp['whh'], pp['bih'], pp['bhh'], 2)
    qs_v = _set2set(x_v, mf_v, pp['wih'], pp['whh'], pp['bih'], pp['bhh'], 2)

    final = jnp.concatenate([qs_u, qs_v], axis=1)
    pred = _pred(final, pp['predwt'], pp['predb'])[:, :1]
    return pred, imap


# speedup vs baseline: 1.3496x; 1.1992x over previous
import functools
import jax
import jax.numpy as jnp
from jax import lax
from jax.experimental import pallas as pl
from jax.experimental.pallas import tpu as pltpu
from jax.experimental.pallas import tpu_sc as plsc

N = 4096
E = 16384
NG = 128
HTRUE = 300
HP = 384
DP = 768
F32 = jnp.float32


NS = 48
RS = 8


def _to_t3(h):
    return h.T.reshape(NS, RS, h.shape[0])


def _proj_kernel(a_ref, w_ref, b_ref, o_ref, t3_ref, *, full, t3):
    acc = jnp.dot(a_ref[...], w_ref[...], preferred_element_type=F32)
    h = jnp.maximum(acc + b_ref[...], 0.0)
    if full:
        o_ref[...] = h
    if t3:
        t3_ref[...] = _to_t3(h)


def _proj(a, wt, b, bm, *, full=True, t3=True):
    m, k = a.shape
    n = wt.shape[1]
    grid = m // bm
    out_specs, out_shape = [], []
    if full:
        out_specs.append(pl.BlockSpec((bm, n), lambda i: (i, 0)))
        out_shape.append(jax.ShapeDtypeStruct((m, n), F32))
    if t3:
        out_specs.append(pl.BlockSpec((NS, RS, bm), lambda i: (0, 0, i)))
        out_shape.append(jax.ShapeDtypeStruct((NS, RS, m), F32))

    def kern(a_ref, w_ref, b_ref, *outs):
        o = outs[0] if full else None
        t = outs[-1] if t3 else None
        _proj_kernel(a_ref, w_ref, b_ref, o, t, full=full, t3=t3)

    res = pl.pallas_call(
        kern,
        grid=(grid,),
        in_specs=[
            pl.BlockSpec((bm, k), lambda i: (i, 0)),
            pl.BlockSpec((k, n), lambda i: (0, 0)),
            pl.BlockSpec((1, n), lambda i: (0, 0)),
        ],
        out_specs=out_specs,
        out_shape=out_shape,
    )(a, wt, b)
    return res if len(res) > 1 else res[0]


def _mlp_kernel(h_ref, a3_ref, p2_ref, w1_ref, b1_ref, w2_ref, b2_ref,
                *outs, l2norm, emit_t, t3):
    bm = h_ref.shape[0]
    a = a3_ref[...]
    p = p2_ref[...]
    psum = p[0:6] + p[6:12] + p[12:18] + p[18:24]
    a = jnp.concatenate([a[:32], a[32:38] + psum, a[38:]], axis=0)
    agg = a.reshape(HP, bm).T
    x = h_ref[...] + agg
    z = jnp.maximum(jnp.dot(x, w1_ref[...], preferred_element_type=F32)
                    + b1_ref[...], 0.0)
    h = jnp.maximum(jnp.dot(z, w2_ref[...], preferred_element_type=F32)
                    + b2_ref[...], 0.0)
    if l2norm:
        nrm = jnp.sqrt(jnp.sum(h * h, axis=1, keepdims=True))
        h = h / jnp.maximum(nrm, 1e-12)
    outs[0][...] = h
    if t3:
        outs[1][...] = _to_t3(h)
    if emit_t:
        outs[-1][...] = h.T


def _mlp(h, a3, p2, w1t, b1, w2t, b2, *, l2norm, emit_t, t3, bm=512):
    m, n = h.shape
    grid = m // bm
    out_shape = [jax.ShapeDtypeStruct((m, n), F32)]
    out_specs = [pl.BlockSpec((bm, n), lambda i: (i, 0))]
    if t3:
        out_shape.append(jax.ShapeDtypeStruct((NS, RS, m), F32))
        out_specs.append(pl.BlockSpec((NS, RS, bm), lambda i: (0, 0, i)))
    if emit_t:
        out_shape.append(jax.ShapeDtypeStruct((n, m), F32))
        out_specs.append(pl.BlockSpec((n, bm), lambda i: (0, i)))
    res = pl.pallas_call(
        functools.partial(_mlp_kernel, l2norm=l2norm, emit_t=emit_t, t3=t3),
        grid=(grid,),
        in_specs=[
            pl.BlockSpec((bm, n), lambda i: (i, 0)),
            pl.BlockSpec((NS, RS, bm), lambda i: (0, 0, i)),
            pl.BlockSpec((24, RS, bm), lambda i: (0, 0, i)),
            pl.BlockSpec((n, n), lambda i: (0, 0)),
            pl.BlockSpec((1, n), lambda i: (0, 0)),
            pl.BlockSpec((n, n), lambda i: (0, 0)),
            pl.BlockSpec((1, n), lambda i: (0, 0)),
        ],
        out_specs=out_specs,
        out_shape=out_shape,
    )(h, a3, p2, w1t, b1, w2t, b2)
    return res if len(res) > 1 else res[0]


_SCN = 512
_NSC = E // _SCN
_ATILES = NS // 2
_LIVE = 38


def _sc_msg_body(h3, e3, src_hbm, dst_hbm, out_hbm, out2_hbm,
                 src_st, dst_st, e_buf, h_slab, acc, sem):
    wid = lax.axis_index("s") * 2 + lax.axis_index("c")

    def _zero_acc():
        def _zero(i, _):
            for r in range(RS):
                acc[r, pl.ds(i * 16, 16)] = jnp.zeros((16,), F32)
            return _
        lax.fori_loop(0, N // 16, _zero, None)

    def _edges(slab, lo, hi):
        pltpu.sync_copy(h3.at[slab], h_slab)

        def _start(cb, b):
            base = cb * _SCN
            pltpu.async_copy(src_hbm.at[pl.ds(base, _SCN)],
                             src_st.at[b], sem.at[b])
            pltpu.async_copy(dst_hbm.at[pl.ds(base, _SCN)],
                             dst_st.at[b], sem.at[b])
            pltpu.async_copy(e3.at[slab, :, pl.ds(base, _SCN)],
                             e_buf.at[b], sem.at[b])

        def _wait(cb, b):
            base = cb * _SCN
            pltpu.make_async_copy(src_hbm.at[pl.ds(base, _SCN)],
                                  src_st.at[b], sem.at[b]).wait()
            pltpu.make_async_copy(dst_hbm.at[pl.ds(base, _SCN)],
                                  dst_st.at[b], sem.at[b]).wait()
            pltpu.make_async_copy(e3.at[slab, :, pl.ds(base, _SCN)],
                                  e_buf.at[b], sem.at[b]).wait()

        _start(lo, lax.rem(lo, 2))

        def _chunk(cb, _):
            b = lax.rem(cb, 2)

            @pl.when(cb + 1 < hi)
            def _():
                _start(cb + 1, 1 - b)
            _wait(cb, b)

            @plsc.parallel_loop(0, _SCN // 16, unroll=2)
            def _grp(j):
                s16 = src_st[b, pl.ds(j * 16, 16)]
                d16 = dst_st[b, pl.ds(j * 16, 16)]
                for r in range(RS):
                    rv = jnp.full((16,), r, jnp.int32)
                    hv = plsc.load_gather(h_slab, [rv, s16])
                    ev = e_buf[b, r, pl.ds(j * 16, 16)]
                    msg = jnp.maximum(hv + ev, 0.0)
                    plsc.addupdate_scatter(acc, [rv, d16], msg)
            return _
        lax.fori_loop(lo, hi, _chunk, None)

    _zero_acc()

    @pl.when(wid < NS - _LIVE)
    def _():
        pltpu.sync_copy(acc, out_hbm.at[_LIVE + wid])

    _edges(wid, 0, _NSC)
    pltpu.sync_copy(acc, out_hbm.at[wid])

    @pl.when((wid >= 6) & (wid < 30))
    def _():
        k = wid - 6
        slab2 = 32 + lax.rem(k, 6)
        q = k // 6
        _zero_acc()

        @pl.when(q == 0)
        def _():
            pltpu.sync_copy(acc, out_hbm.at[slab2])
        nq = _NSC // 4
        _edges(slab2, q * nq, (q + 1) * nq)
        pltpu.sync_copy(acc, out2_hbm.at[k])


def _sc_msg(h3, e3, src, dst):
    mesh = plsc.VectorSubcoreMesh(core_axis_name="c", subcore_axis_name="s")
    return pl.kernel(
        _sc_msg_body,
        out_type=[jax.ShapeDtypeStruct((NS, RS, N), F32),
                  jax.ShapeDtypeStruct((24, RS, N), F32)],
        mesh=mesh,
        compiler_params=pltpu.CompilerParams(needs_layout_passes=False),
        scratch_types=[
            pltpu.VMEM((2, _SCN), jnp.int32),
            pltpu.VMEM((2, _SCN), jnp.int32),
            pltpu.VMEM((2, RS, _SCN), F32),
            pltpu.VMEM((RS, N), F32),
            pltpu.VMEM((RS, N), F32),
            pltpu.SemaphoreType.DMA((2,)),
        ],
    )(h3, e3, src, dst)


def _inter_kernel(u_ref, vt_ref, v_ref, bu_ref, bv_ref,
                  imap_ref, up_ref, vp_ref):
    i = pl.program_id(0)

    @pl.when(i == 0)
    def _():
        vp_ref[...] = jnp.zeros_like(vp_ref)

    imap_ref[...] = jnp.zeros_like(imap_ref)

    bu = bu_ref[...]
    bv = bv_ref[...]
    blo = jnp.min(bu)
    bhi = jnp.max(bu)
    start = jnp.sum((bv < blo).astype(F32)).astype(jnp.int32)
    end = jnp.sum((bv <= bhi).astype(F32)).astype(jnp.int32)
    bm = 128
    jlo = start // bm
    jhi = (end + bm - 1) // bm

    u_blk = u_ref[...]

    def _col(j, acc):
        c0 = j * bm
        s = jnp.dot(u_blk, vt_ref[:, pl.ds(c0, bm)],
                    preferred_element_type=F32)
        mask = bu == bv_ref[0:1, pl.ds(c0, bm)]
        s = jnp.where(mask, s, 0.0)
        imap_ref[:, pl.ds(c0, bm)] = s
        acc = acc + jnp.dot(s, v_ref[pl.ds(c0, bm), :],
                            preferred_element_type=F32)
        vp_ref[pl.ds(c0, bm), :] += lax.dot_general(
            s, u_blk, (((0,), (0,)), ((), ())),
            preferred_element_type=F32)
        return acc
    up_ref[...] = lax.fori_loop(jlo, jhi, _col, jnp.zeros((bm, HP), F32))


def _interact(u, v, vt, bu, bv):
    bm = 128
    grid = N // bm
    return pl.pallas_call(
        _inter_kernel,
        grid=(grid,),
        in_specs=[
            pl.BlockSpec((bm, HP), lambda i: (i, 0)),
            pl.BlockSpec((HP, N), lambda i: (0, 0)),
            pl.BlockSpec((N, HP), lambda i: (0, 0)),
            pl.BlockSpec((bm, 1), lambda i: (i, 0)),
            pl.BlockSpec((1, N), lambda i: (0, 0)),
        ],
        out_specs=[
            pl.BlockSpec((bm, N), lambda i: (i, 0)),
            pl.BlockSpec((bm, HP), lambda i: (i, 0)),
            pl.BlockSpec((N, HP), lambda i: (0, 0)),
        ],
        out_shape=[
            jax.ShapeDtypeStruct((N, N), F32),
            jax.ShapeDtypeStruct((N, HP), F32),
            jax.ShapeDtypeStruct((N, HP), F32),
        ],
    )(u, vt, v, bu, bv)


def _lstm_kernel(q_ref, h_ref, c_ref, wih_ref, whh_ref, bih_ref, bhh_ref,
                 ho_ref, co_ref):
    gates = (jnp.dot(q_ref[...], wih_ref[...], preferred_element_type=F32)
             + bih_ref[...]
             + jnp.dot(h_ref[...], whh_ref[...], preferred_element_type=F32)
             + bhh_ref[...])
    ig = jax.nn.sigmoid(gates[:, 0 * DP:1 * DP])
    fg = jax.nn.sigmoid(gates[:, 1 * DP:2 * DP])
    gg = jnp.tanh(gates[:, 2 * DP:3 * DP])
    og = jax.nn.sigmoid(gates[:, 3 * DP:4 * DP])
    cc = fg * c_ref[...] + ig * gg
    ho_ref[...] = og * jnp.tanh(cc)
    co_ref[...] = cc


def _s2s_node_kernel(x_ref, mf_ref, q_ref, qs_ref):
    x = x_ref[...]
    mf = mf_ref[...]
    q = q_ref[...]
    escore = lax.dot_general(x, q, (((1,), (1,)), ((), ())),
                             preferred_element_type=F32,
                             precision=jax.lax.Precision.HIGHEST)
    e_node = jnp.sum(escore * mf, axis=1, keepdims=True)
    em = e_node * mf + (mf - 1.0) * 1e30
    emax = jnp.max(em, axis=0, keepdims=True)
    e_max_node = jnp.sum(mf * emax, axis=1, keepdims=True)
    a = jnp.exp(e_node - e_max_node)
    asum = jnp.sum(a * mf, axis=0, keepdims=True)
    asum_node = jnp.sum(mf * asum, axis=1, keepdims=True)
    a = a / (asum_node + 1e-16)
    r = lax.dot_general(mf, a * x, (((0,), (0,)), ((), ())),
                        preferred_element_type=F32)
    qs_ref[...] = jnp.concatenate([q, r], axis=1)


def _set2set(x, mf, wih, whh, bih, bhh, steps):
    q_star = jnp.zeros((NG, 2 * DP), F32)
    hh = jnp.zeros((NG, DP), F32)
    cc = jnp.zeros((NG, DP), F32)
    for _ in range(steps):
        hh, cc = pl.pallas_call(
            _lstm_kernel,
            out_shape=[jax.ShapeDtypeStruct((NG, DP), F32),
                       jax.ShapeDtypeStruct((NG, DP), F32)],
        )(q_star, hh, cc, wih, whh, bih, bhh)
        q_star = pl.pallas_call(
            _s2s_node_kernel,
            out_shape=jax.ShapeDtypeStruct((NG, 2 * DP), F32),
        )(x, mf, hh)
    return q_star


def _pred_kernel(f_ref, w_ref, b_ref, o_ref):
    o_ref[...] = (jnp.dot(f_ref[...], w_ref[...], preferred_element_type=F32)
                  + b_ref[...])


def _pred(f, wt, b):
    return pl.pallas_call(
        _pred_kernel,
        out_shape=jax.ShapeDtypeStruct((NG, 128), F32),
    )(f, wt, b)


def _padt(w, rows, cols):
    wt = w.T
    return jnp.zeros((rows, cols), F32).at[:wt.shape[0], :wt.shape[1]].set(wt)


def _pad_vec(b, n):
    return jnp.zeros((1, n), F32).at[0, :b.shape[0]].set(b)


def _prep(params):
    p = {}
    p['w0t'] = _padt(params['lin0_w'], 133, HP)
    p['b0'] = _pad_vec(params['lin0_b'], HP)
    p['wet'] = _padt(params['edge_w'], 14, HP)
    p['be'] = _pad_vec(params['edge_b'], HP)
    for i in range(3):
        p['w1t%d' % i] = _padt(params['mlp1_w_%d' % i], HP, HP)
        p['b1%d' % i] = _pad_vec(params['mlp1_b_%d' % i], HP)
        p['w2t%d' % i] = _padt(params['mlp2_w_%d' % i], HP, HP)
        p['b2%d' % i] = _pad_vec(params['mlp2_b_%d' % i], HP)
    half = ((0, 0), (300, 384))
    qs_half = ((0, 0), (300, 384), (600, DP), (900, DP + 384))
    wih = params['lstm_wih']
    whh = params['lstm_whh']
    wih_p = jnp.zeros((2 * DP, 4 * DP), F32)
    whh_p = jnp.zeros((DP, 4 * DP), F32)
    bih_p = jnp.zeros((1, 4 * DP), F32)
    bhh_p = jnp.zeros((1, 4 * DP), F32)
    for g in range(4):
        for (to, po) in half:
            ro = g * 600 + to
            co = g * DP + po
            for (ti, pi) in qs_half:
                wih_p = wih_p.at[pi:pi + 300, co:co + 300].set(
                    wih[ro:ro + 300, ti:ti + 300].T)
            for (ti, pi) in half:
                whh_p = whh_p.at[pi:pi + 300, co:co + 300].set(
                    whh[ro:ro + 300, ti:ti + 300].T)
            bih_p = bih_p.at[0, co:co + 300].set(
                params['lstm_bih'][ro:ro + 300])
            bhh_p = bhh_p.at[0, co:co + 300].set(
                params['lstm_bhh'][ro:ro + 300])
    p['wih'] = wih_p
    p['whh'] = whh_p
    p['bih'] = bih_p
    p['bhh'] = bhh_p
    pw = params['pred_w'][0]
    pwt = jnp.zeros((4 * DP, 128), F32)
    for g in range(4):
        for (to, po) in half:
            pwt = pwt.at[g * DP + po:g * DP + po + 300, 0].set(
                pw[g * 600 + to:g * 600 + to + 300])
    p['predwt'] = pwt
    p['predb'] = jnp.zeros((1, 128), F32).at[0, 0].set(params['pred_b'][0])
    return p


def _gine_side(x, eattr, src, dst, pp, *, emit_t):
    h, h3 = _proj(x, pp['w0t'], pp['b0'], 512)
    e3 = _proj(eattr, pp['wet'], pp['be'], 1024, full=False)
    for i in range(3):
        a3, p2 = _sc_msg(h3, e3, src, dst)
        last = (i == 2)
        res = _mlp(h, a3, p2,
                   pp['w1t%d' % i], pp['b1%d' % i],
                   pp['w2t%d' % i], pp['b2%d' % i],
                   l2norm=last, emit_t=(last and emit_t), t3=not last)
        if last:
            return res
        h, h3 = res


def kernel(solute_x, solute_edge_index, solute_edge_attr, solute_batch,
           solute_len, solvent_x, solvent_edge_index, solvent_edge_attr,
           solvent_batch, solvent_len, params):
    pp = _prep(params)
    src_u = solute_edge_index[0].astype(jnp.int32)
    dst_u = solute_edge_index[1].astype(jnp.int32)
    src_v = solvent_edge_index[0].astype(jnp.int32)
    dst_v = solvent_edge_index[1].astype(jnp.int32)

    u = _gine_side(solute_x, solute_edge_attr, src_u, dst_u, pp, emit_t=False)
    v, vt = _gine_side(solvent_x, solvent_edge_attr, src_v, dst_v, pp,
                       emit_t=True)

    bu = solute_batch.astype(F32).reshape(N, 1)
    bv = solvent_batch.astype(F32).reshape(1, N)
    imap, u_p, v_p = _interact(u, v, vt, bu, bv)

    x_u = jnp.concatenate([u, u_p], axis=1)
    x_v = jnp.concatenate([v, v_p], axis=1)
    mf_u = solute_len.T
    mf_v = solvent_len.T
    qs_u = _set2set(x_u, mf_u, pp['wih'], pp['whh'], pp['bih'], pp['bhh'], 2)
    qs_v = _set2set(x_v, mf_v, pp['wih'], pp['whh'], pp['bih'], pp['bhh'], 2)

    final = jnp.concatenate([qs_u, qs_v], axis=1)
    pred = _pred(final, pp['predwt'], pp['predb'])[:, :1]
    return pred, imap
